# baseline (speedup 1.0000x reference)
import functools

import jax
import jax.numpy as jnp
from jax import lax
from jax.experimental import pallas as pl
from jax.experimental.pallas import tpu as pltpu
from jax.experimental.pallas import tpu_sc as plsc

N = 5000
NPAD = 5120
COLS = NPAD // 16
NGROUP = COLS // 16
NCHUNKS = NPAD // 16
INV_CELL = 0.03125
INV_YCELL = 0.015625
GX = 16
GY = 8
NCELL = GX * GY


def _make_sc_call():
    mesh = plsc.VectorSubcoreMesh(core_axis_name="c", subcore_axis_name="s")

    @functools.partial(
        pl.kernel,
        mesh=mesh,
        compiler_params=pltpu.CompilerParams(needs_layout_passes=False),
        out_type=jax.ShapeDtypeStruct((2 * NPAD,), jnp.int32),
        scratch_types=[
            pltpu.VMEM((5, NPAD), jnp.float32),
            pltpu.VMEM((NPAD,), jnp.float32),
            pltpu.VMEM((NPAD,), jnp.float32),
            pltpu.VMEM((NPAD,), jnp.float32),
            pltpu.VMEM((NPAD,), jnp.float32),
            pltpu.VMEM((NPAD,), jnp.float32),
            pltpu.VMEM((NPAD,), jnp.float32),
            pltpu.VMEM((NPAD,), jnp.int32),
            pltpu.VMEM((NPAD,), jnp.int32),
            pltpu.VMEM((4096,), jnp.int32),
            pltpu.VMEM((COLS,), jnp.int32),
            pltpu.VMEM((COLS,), jnp.int32),
            pltpu.SMEM((530,), jnp.int32),
            pltpu.VMEM_SHARED((NPAD,), jnp.int32),
        ],
    )
    def sc_kernel(data_hbm, out_hbm, data_v, sx1_v, sy1_v, sx2_v, sy2_v,
                  ss_v, sarea_v, sidx_v, cell_v, hist_v, keep_v, kidx_v,
                  sm, sh_o):
        sort_rows = (sx1_v, sy1_v, sx2_v, sy2_v, ss_v)
        b = lax.axis_index("c")
        w = lax.axis_index("s")

        pltpu.sync_copy(data_hbm.at[b], data_v)

        iota16 = lax.broadcasted_iota(jnp.int32, (16,), 0)
        zeros16 = jnp.zeros((16,), jnp.int32)
        ones16 = jnp.ones((16,), jnp.int32)

        def zk(t, _):
            keep_v[pl.ds(t * 16, 16)] = zeros16
            return 0
        lax.fori_loop(0, NGROUP, zk, 0)
        pltpu.sync_copy(keep_v, sh_o.at[pl.ds(w * COLS, COLS)])

        def zh(t, _):
            hist_v[pl.ds(t * 16, 16)] = zeros16
            return 0
        lax.fori_loop(0, NCELL, zh, 0)

        def cells_body(v, _):
            sl = pl.ds(v * 16, 16)
            cy = (data_v[1, sl] * jnp.float32(INV_YCELL)).astype(jnp.int32)
            cx = (data_v[0, sl] * jnp.float32(INV_CELL)).astype(jnp.int32)
            cell = cy * GX + cx
            cell_v[sl] = cell
            plsc.addupdate_scatter(hist_v, [cell * 16 + iota16], ones16)
            return 0
        lax.fori_loop(0, NCHUNKS, cells_body, 0)

        def pfx_body(c, run):
            tot = plsc.cumsum(hist_v[pl.ds(c * 16, 16)])[15]
            sm[c] = run
            sm[256 + c] = run
            return run + tot
        run = lax.fori_loop(0, NCELL, pfx_body, jnp.int32(0))
        sm[256 + NCELL] = run

        def place_body(v, _):
            off = v * 16
            sl = pl.ds(off, 16)
            cells = cell_v[sl]
            posv = zeros16
            for l in range(16):
                c = cells[l]
                p = sm[c]
                sm[c] = p + 1
                posv = jnp.where(iota16 == l, p, posv)
            for r in range(5):
                plsc.store_scatter(sort_rows[r], [posv], data_v[r, sl])
            plsc.store_scatter(sidx_v, [posv], iota16 + off)
            return 0
        lax.fori_loop(0, NCHUNKS, place_body, 0)

        def area_body(v, _):
            sl = pl.ds(v * 16, 16)
            sarea_v[sl] = (sx2_v[sl] - sx1_v[sl]) * (sy2_v[sl] - sy1_v[sl])
            return 0
        lax.fori_loop(0, NCHUNKS, area_body, 0)

        def group_body(g, _):
            base = w * COLS + g * 16
            gsl = pl.ds(base, 16)
            cx1 = sx1_v[gsl]
            cy1 = sy1_v[gsl]
            cx2 = sx2_v[gsl]
            cy2 = sy2_v[gsl]
            cs = ss_v[gsl]
            ca = sarea_v[gsl]
            corig = sidx_v[gsl]

            fzero = jnp.float32(0.0)
            guard = jnp.float32(65.5)
            inv = jnp.float32(INV_CELL)
            keepvec = zeros16
            for l in range(16):
                x1j = cx1[l]
                y1j = cy1[l]
                x2j = cx2[l]
                y2j = cy2[l]
                sj = cs[l]
                aj = ca[l]
                jo = corig[l]
                sxlo = (jnp.maximum(x1j - guard, fzero) * inv).astype(jnp.int32)
                sxhi = jnp.minimum((x2j * inv).astype(jnp.int32), GX - 1)
                invy = jnp.float32(INV_YCELL)
                sylo = (jnp.maximum(y1j - guard, fzero) * invy).astype(jnp.int32)
                syhi = jnp.minimum((y2j * invy).astype(jnp.int32), GY - 1)

                def sup_chunk(st):
                    sl = pl.ds(st, 16)
                    ltx = jnp.maximum(sx1_v[sl], x1j)
                    lty = jnp.maximum(sy1_v[sl], y1j)
                    rbx = jnp.minimum(sx2_v[sl], x2j)
                    rby = jnp.minimum(sy2_v[sl], y2j)
                    wd = jnp.maximum(rbx - ltx, fzero)
                    ht = jnp.maximum(rby - lty, fzero)
                    inter = wd * ht
                    union = (sarea_v[sl] + aj) - inter
                    iou = inter / union
                    m = iou >= jnp.float32(0.7)
                    vv = jnp.where(m, ss_v[sl], fzero)
                    sup = (vv > sj) | ((vv == sj) & (sidx_v[sl] < jo))
                    return jnp.where(sup, 1, 0).astype(jnp.int32)

                def strip_body(s, acc):
                    off_lo = sm[256 + s * GX + sxlo]
                    off_hi = sm[256 + s * GX + sxhi + 1]
                    npair = jnp.where(
                        s <= syhi, (off_hi - off_lo + 31) // 32, 0)

                    def chunk_body(k, acc2):
                        st0 = jnp.minimum(off_lo + k * 32, NPAD - 16)
                        st1 = jnp.minimum(off_lo + k * 32 + 16, NPAD - 16)
                        return acc2 | sup_chunk(st0) | sup_chunk(st1)

                    return lax.fori_loop(0, npair, chunk_body, acc)

                acc = zeros16
                for soff in range(4):
                    acc = strip_body(jnp.minimum(sylo + soff, GY - 1), acc)
                pc = plsc.all_reduce_population_count(acc > 0)
                keep_l = jnp.where(pc[0] == 0, 1, 0).astype(jnp.int32)
                keepvec = jnp.where(iota16 == l, keep_l, keepvec)

            keep_v[pl.ds(g * 16, 16)] = keepvec
            kidx_v[pl.ds(g * 16, 16)] = corig
            return 0

        plsc.subcore_barrier()
        lax.fori_loop(0, NGROUP, group_body, 0)

        pltpu.sync_copy(keep_v, sh_o.at[kidx_v], add=True)
        plsc.subcore_barrier()

        @pl.when(w == 0)
        def _():
            pltpu.sync_copy(sh_o, cell_v)
            pltpu.sync_copy(cell_v, out_hbm.at[pl.ds(b * NPAD, NPAD)])

    return sc_kernel


_sc_call = _make_sc_call()


def kernel(box, score):
    data = jnp.concatenate([box, score], axis=1)
    data = jnp.pad(data, ((0, 0), (0, 0), (0, NPAD - N)))
    out = _sc_call(data).reshape(2, NPAD)
    return out[:, :N].astype(jnp.int64)


# speedup vs baseline: 1.6505x; 1.6505x over previous
import functools

import jax
import jax.numpy as jnp
from jax import lax
from jax.experimental import pallas as pl
from jax.experimental.pallas import tpu as pltpu
from jax.experimental.pallas import tpu_sc as plsc

N = 5000
NPAD = 5120
COLS = NPAD // 16
NGROUP = COLS // 16
NCHUNKS = NPAD // 16
INV_CELL = 0.03125
INV_YCELL = 0.015625
GX = 16
GY = 8
NCELL = GX * GY


def _make_sc_call():
    mesh = plsc.VectorSubcoreMesh(core_axis_name="c", subcore_axis_name="s")

    @functools.partial(
        pl.kernel,
        mesh=mesh,
        compiler_params=pltpu.CompilerParams(needs_layout_passes=False),
        out_type=jax.ShapeDtypeStruct((2 * NPAD,), jnp.int32),
        scratch_types=[
            pltpu.VMEM((5, NPAD), jnp.float32),
            pltpu.VMEM((NPAD,), jnp.float32),
            pltpu.VMEM((NPAD,), jnp.float32),
            pltpu.VMEM((NPAD,), jnp.float32),
            pltpu.VMEM((NPAD,), jnp.float32),
            pltpu.VMEM((NPAD,), jnp.float32),
            pltpu.VMEM((NPAD,), jnp.float32),
            pltpu.VMEM((NPAD,), jnp.int32),
            pltpu.VMEM((NPAD,), jnp.int32),
            pltpu.VMEM((4096,), jnp.int32),
            pltpu.VMEM((COLS,), jnp.int32),
            pltpu.VMEM((COLS,), jnp.int32),
            pltpu.SMEM((530,), jnp.int32),
            pltpu.VMEM_SHARED((NPAD,), jnp.int32),
        ],
    )
    def sc_kernel(data_hbm, out_hbm, data_v, sx1_v, sy1_v, sx2_v, sy2_v,
                  ss_v, sarea_v, sidx_v, cell_v, hist_v, keep_v, kidx_v,
                  sm, sh_o):
        sort_rows = (sx1_v, sy1_v, sx2_v, sy2_v, ss_v)
        b = lax.axis_index("c")
        w = lax.axis_index("s")

        pltpu.sync_copy(data_hbm.at[b], data_v)

        iota16 = lax.broadcasted_iota(jnp.int32, (16,), 0)
        zeros16 = jnp.zeros((16,), jnp.int32)
        ones16 = jnp.ones((16,), jnp.int32)

        def zk(t, _):
            keep_v[pl.ds(t * 16, 16)] = zeros16
            return 0
        lax.fori_loop(0, NGROUP, zk, 0)
        pltpu.sync_copy(keep_v, sh_o.at[pl.ds(w * COLS, COLS)])

        def zh(t, _):
            hist_v[pl.ds(t * 16, 16)] = zeros16
            return 0
        lax.fori_loop(0, NCELL, zh, 0)

        def cells_body(v, _):
            sl = pl.ds(v * 16, 16)
            cy = (data_v[1, sl] * jnp.float32(INV_YCELL)).astype(jnp.int32)
            cx = (data_v[0, sl] * jnp.float32(INV_CELL)).astype(jnp.int32)
            cell = cy * GX + cx
            cell_v[sl] = cell
            plsc.addupdate_scatter(hist_v, [cell * 16 + iota16], ones16)
            return 0
        lax.fori_loop(0, NCHUNKS, cells_body, 0)

        def pfx_body(c, run):
            tot = plsc.cumsum(hist_v[pl.ds(c * 16, 16)])[15]
            sm[c] = run
            sm[256 + c] = run
            return run + tot
        run = lax.fori_loop(0, NCELL, pfx_body, jnp.int32(0))
        sm[256 + NCELL] = run

        def place_body(v, _):
            off = v * 16
            sl = pl.ds(off, 16)
            cells = cell_v[sl]
            posv = zeros16
            for l in range(16):
                c = cells[l]
                p = sm[c]
                sm[c] = p + 1
                posv = jnp.where(iota16 == l, p, posv)
            for r in range(5):
                plsc.store_scatter(sort_rows[r], [posv], data_v[r, sl])
            plsc.store_scatter(sidx_v, [posv], iota16 + off)
            return 0
        lax.fori_loop(0, NCHUNKS, place_body, 0)

        def area_body(v, _):
            sl = pl.ds(v * 16, 16)
            sarea_v[sl] = (sx2_v[sl] - sx1_v[sl]) * (sy2_v[sl] - sy1_v[sl])
            return 0
        lax.fori_loop(0, NCHUNKS, area_body, 0)

        def group_body(g, _):
            base = w * COLS + g * 16
            gsl = pl.ds(base, 16)
            cx1 = sx1_v[gsl]
            cy1 = sy1_v[gsl]
            cx2 = sx2_v[gsl]
            cy2 = sy2_v[gsl]
            cs = ss_v[gsl]
            ca = sarea_v[gsl]
            corig = sidx_v[gsl]

            fzero = jnp.float32(0.0)
            guard = jnp.float32(65.5)
            inv = jnp.float32(INV_CELL)
            keepvec = zeros16
            for l in range(16):
                x1j = cx1[l]
                y1j = cy1[l]
                x2j = cx2[l]
                y2j = cy2[l]
                sj = cs[l]
                aj = ca[l]
                jo = corig[l]
                sxlo = (jnp.maximum(x1j - guard, fzero) * inv).astype(jnp.int32)
                sxhi = jnp.minimum((x2j * inv).astype(jnp.int32), GX - 1)
                invy = jnp.float32(INV_YCELL)
                sylo = (jnp.maximum(y1j - guard, fzero) * invy).astype(jnp.int32)
                syhi = jnp.minimum((y2j * invy).astype(jnp.int32), GY - 1)

                def sup_chunk(st):
                    sl = pl.ds(st, 16)
                    ltx = jnp.maximum(sx1_v[sl], x1j)
                    lty = jnp.maximum(sy1_v[sl], y1j)
                    rbx = jnp.minimum(sx2_v[sl], x2j)
                    rby = jnp.minimum(sy2_v[sl], y2j)
                    wd = jnp.maximum(rbx - ltx, fzero)
                    ht = jnp.maximum(rby - lty, fzero)
                    inter = wd * ht
                    union = (sarea_v[sl] + aj) - inter
                    iou = inter / union
                    m = iou >= jnp.float32(0.7)
                    vv = jnp.where(m, ss_v[sl], fzero)
                    sup = (vv > sj) | ((vv == sj) & (sidx_v[sl] < jo))
                    return jnp.where(sup, 1, 0).astype(jnp.int32)

                def strip_body(s, acc):
                    off_lo = sm[256 + s * GX + sxlo]
                    off_hi = sm[256 + s * GX + sxhi + 1]
                    npair = (off_hi - off_lo + 31) // 32

                    def chunk_body(k, acc2):
                        st0 = jnp.minimum(off_lo + k * 32, NPAD - 16)
                        st1 = jnp.minimum(off_lo + k * 32 + 16, NPAD - 16)
                        return acc2 | sup_chunk(st0) | sup_chunk(st1)

                    return lax.fori_loop(0, npair, chunk_body, acc)

                acc = lax.fori_loop(sylo, syhi + 1, strip_body, zeros16)
                pc = plsc.all_reduce_population_count(acc > 0)
                keep_l = jnp.where(pc[0] == 0, 1, 0).astype(jnp.int32)
                keepvec = jnp.where(iota16 == l, keep_l, keepvec)

            keep_v[pl.ds(g * 16, 16)] = keepvec
            kidx_v[pl.ds(g * 16, 16)] = corig
            return 0

        plsc.subcore_barrier()
        lax.fori_loop(0, NGROUP, group_body, 0)

        pltpu.sync_copy(keep_v, sh_o.at[kidx_v], add=True)
        plsc.subcore_barrier()

        @pl.when(w == 0)
        def _():
            pltpu.sync_copy(sh_o, cell_v)
            pltpu.sync_copy(cell_v, out_hbm.at[pl.ds(b * NPAD, NPAD)])

    return sc_kernel


_sc_call = _make_sc_call()


def kernel(box, score):
    data = jnp.concatenate([box, score], axis=1)
    data = jnp.pad(data, ((0, 0), (0, 0), (0, NPAD - N)))
    out = _sc_call(data).reshape(2, NPAD)
    return out[:, :N].astype(jnp.int64)
